# baseline (speedup 1.0000x reference)
import functools

import jax
import jax.numpy as jnp
from jax import lax
from jax.experimental import pallas as pl
from jax.experimental.pallas import tpu as pltpu
from jax.experimental.pallas import tpu_sc as plsc

N = 10000
E = 160000
D = 256
DH = 128
NP = 10240
CH = 120
EPT = E // 16
EPD = E // 32
RPT = NP // 16
PADE = 5040

_sc_mesh = plsc.VectorSubcoreMesh(core_axis_name="c", subcore_axis_name="s")


def _chunks(total):
    out = []
    o = 0
    while o < total:
        sz = min(CH, total - o)
        out.append((o, sz))
        o += sz
    return out


def _deg_body(ei1, deg_out, idxf, ones, zbuf, deg_sh, sem):
    c = lax.axis_index("c")
    s = lax.axis_index("s")
    wid = s * 2 + c

    for k in range(8):
        ones[pl.ds(k * 16, 16)] = jnp.ones((16,), jnp.float32)

    def _z(i, _):
        zbuf[pl.ds(i * 16, 16)] = jnp.zeros((16,), jnp.float32)
        return 0

    lax.fori_loop(0, RPT // 16, _z, 0)
    pltpu.sync_copy(zbuf, deg_sh.at[pl.ds(s * RPT, RPT)])
    plsc.subcore_barrier()

    pltpu.sync_copy(ei1.at[pl.ds(E + wid * EPD, EPD)], idxf)
    descs = [
        pltpu.async_copy(ones.at[pl.ds(0, sz)],
                         deg_sh.at[idxf.at[pl.ds(o, sz)]], sem, add=True)
        for o, sz in _chunks(EPD)
    ]
    for d in descs:
        d.wait()
    plsc.subcore_barrier()
    pltpu.sync_copy(deg_sh.at[pl.ds(s * RPT, RPT)],
                    deg_out.at[c, pl.ds(s * RPT, RPT)])


_deg_kernel = functools.partial(
    pl.kernel,
    out_type=jax.ShapeDtypeStruct((2, NP), jnp.float32),
    mesh=_sc_mesh,
    scratch_types=[
        pltpu.VMEM((EPD,), jnp.int32),
        pltpu.VMEM((128,), jnp.float32),
        pltpu.VMEM((RPT,), jnp.float32),
        pltpu.VMEM_SHARED((NP,), jnp.float32),
        pltpu.SemaphoreType.DMA,
    ],
)(_deg_body)


def _agg_body(ei1, ycat, acc_out, dstf, srcf, buf0, buf1, acc_sh,
              sem0, sem1):
    c = lax.axis_index("c")
    s = lax.axis_index("s")

    pltpu.sync_copy(ycat.at[pl.ds(c * NP + s * RPT, RPT)],
                    acc_sh.at[pl.ds(s * RPT, RPT)])
    plsc.subcore_barrier()

    off = c * NP
    gsems = (sem0, sem1)
    bufs = (buf0, buf1)
    half = EPT // 2
    halfp = PADE
    for p in range(2):
        for k in range(PADE // 16 - 3, PADE // 16):
            lane = lax.broadcasted_iota(jnp.int32, (16,), 0) + k * 16
            srcf[pl.ds(k * 16, 16)] = lane - (PADE - 64)
            dstf[pl.ds(k * 16, 16)] = lane - (PADE - 64) + N
        pltpu.sync_copy(ei1.at[pl.ds(s * EPT + p * half, half)], dstf.at[pl.ds(0, half)])
        pltpu.sync_copy(ei1.at[pl.ds(E + s * EPT + p * half, half)], srcf.at[pl.ds(0, half)])

        def _adj(i, _):
            v = srcf[pl.ds(i * 16, 16)]
            srcf[pl.ds(i * 16, 16)] = v + off
            return 0

        lax.fori_loop(0, halfp // 16, _adj, 0)

        nchunk = halfp // CH
        pend = pltpu.async_copy(ycat.at[srcf.at[pl.ds(0, CH)]], buf0,
                                gsems[0])
        for j in range(nchunk):
            nxt = None
            if j + 1 < nchunk:
                nxt = pltpu.async_copy(
                    ycat.at[srcf.at[pl.ds((j + 1) * CH, CH)]],
                    bufs[(j + 1) % 2], gsems[(j + 1) % 2])
            pend.wait()
            pltpu.sync_copy(bufs[j % 2],
                            acc_sh.at[dstf.at[pl.ds(j * CH, CH)]], add=True)
            pend = nxt

    plsc.subcore_barrier()
    pltpu.sync_copy(acc_sh.at[pl.ds(s * RPT, RPT)],
                    acc_out.at[c, pl.ds(s * RPT, RPT), :])


_agg_kernel = functools.partial(
    pl.kernel,
    out_type=jax.ShapeDtypeStruct((2, NP, DH), jnp.float32),
    mesh=_sc_mesh,
    scratch_types=[
        pltpu.VMEM((PADE,), jnp.int32),
        pltpu.VMEM((PADE,), jnp.int32),
        pltpu.VMEM((CH, DH), jnp.float32),
        pltpu.VMEM((CH, DH), jnp.float32),
        pltpu.VMEM_SHARED((NP, DH), jnp.float32),
        pltpu.SemaphoreType.DMA,
        pltpu.SemaphoreType.DMA,
    ],
)(_agg_body)


def _lin_body(x_ref, w_ref, b_ref, deg_ref, y_ref):
    xl = lax.dot_general(x_ref[...], w_ref[...], (((1,), (1,)), ((), ())),
                         preferred_element_type=jnp.float32)
    xl = xl + b_ref[...]
    dinv = lax.rsqrt(deg_ref[0, :] + deg_ref[1, :] + 1.0)[:, None]
    y = xl * dinv
    y_ref[0] = y[:, :DH]
    y_ref[1] = y[:, DH:]


def _lin_call(x, w, b2, deg_part):
    blk = NP // 8
    return pl.pallas_call(
        _lin_body,
        grid=(8,),
        in_specs=[
            pl.BlockSpec((blk, D), lambda i: (i, 0)),
            pl.BlockSpec((D, D), lambda i: (0, 0)),
            pl.BlockSpec((1, D), lambda i: (0, 0)),
            pl.BlockSpec((2, blk), lambda i: (0, i)),
        ],
        out_specs=pl.BlockSpec((2, blk, DH), lambda i: (0, i, 0)),
        out_shape=jax.ShapeDtypeStruct((2, NP, DH), jnp.float32),
    )(x, w, b2, deg_part)


def _out_body(acc_ref, deg_ref, o_ref):
    dinv = lax.rsqrt(deg_ref[0, :] + deg_ref[1, :] + 1.0)[:, None]
    o_ref[...] = jnp.concatenate([acc_ref[0] * dinv, acc_ref[1] * dinv],
                                 axis=1)


def _out_call(acc, deg_part):
    blk = NP // 8
    return pl.pallas_call(
        _out_body,
        grid=(8,),
        in_specs=[
            pl.BlockSpec((2, blk, DH), lambda i: (0, i, 0)),
            pl.BlockSpec((2, blk), lambda i: (0, i)),
        ],
        out_specs=pl.BlockSpec((blk, D), lambda i: (i, 0)),
        out_shape=jax.ShapeDtypeStruct((N, D), jnp.float32),
    )(acc, deg_part)


def kernel(x, edge_index, W, b):
    ei1 = edge_index.reshape(2 * E)
    deg_part = _deg_kernel(ei1)
    y = _lin_call(x, W, b.reshape(1, D), deg_part)
    ycat = y.reshape(2 * NP, DH)
    acc = _agg_kernel(ei1, ycat)
    return _out_call(acc, deg_part)


# speedup vs baseline: 1.0113x; 1.0113x over previous
import functools

import jax
import jax.numpy as jnp
from jax import lax
from jax.experimental import pallas as pl
from jax.experimental.pallas import tpu as pltpu
from jax.experimental.pallas import tpu_sc as plsc

N = 10000
E = 160000
D = 256
DH = 128
NP = 10240
CH = 120
EPT = E // 16
EPD = E // 32
RPT = NP // 16
PADE = 5040

_sc_mesh = plsc.VectorSubcoreMesh(core_axis_name="c", subcore_axis_name="s")


def _chunks(total):
    out = []
    o = 0
    while o < total:
        sz = min(CH, total - o)
        out.append((o, sz))
        o += sz
    return out


def _deg_body(ei1, deg_out, idxf, ones, zbuf, deg_sh, sem):
    c = lax.axis_index("c")
    s = lax.axis_index("s")
    wid = s * 2 + c

    for k in range(8):
        ones[pl.ds(k * 16, 16)] = jnp.ones((16,), jnp.float32)

    def _z(i, _):
        zbuf[pl.ds(i * 16, 16)] = jnp.zeros((16,), jnp.float32)
        return 0

    lax.fori_loop(0, RPT // 16, _z, 0)
    pltpu.sync_copy(zbuf, deg_sh.at[pl.ds(s * RPT, RPT)])
    plsc.subcore_barrier()

    pltpu.sync_copy(ei1.at[pl.ds(E + wid * EPD, EPD)], idxf)
    descs = [
        pltpu.async_copy(ones.at[pl.ds(0, sz)],
                         deg_sh.at[idxf.at[pl.ds(o, sz)]], sem, add=True)
        for o, sz in _chunks(EPD)
    ]
    for d in descs:
        d.wait()
    plsc.subcore_barrier()
    pltpu.sync_copy(deg_sh.at[pl.ds(s * RPT, RPT)],
                    deg_out.at[c, pl.ds(s * RPT, RPT)])


_deg_kernel = functools.partial(
    pl.kernel,
    out_type=jax.ShapeDtypeStruct((2, NP), jnp.float32),
    mesh=_sc_mesh,
    scratch_types=[
        pltpu.VMEM((EPD,), jnp.int32),
        pltpu.VMEM((128,), jnp.float32),
        pltpu.VMEM((RPT,), jnp.float32),
        pltpu.VMEM_SHARED((NP,), jnp.float32),
        pltpu.SemaphoreType.DMA,
    ],
)(_deg_body)


def _agg_body(ei1, ycat, acc_out, dstf, srcf, buf0, buf1, acc_sh,
              sem0, sem1):
    c = lax.axis_index("c")
    s = lax.axis_index("s")

    off = c * NP
    gsems = (sem0, sem1)
    bufs = (buf0, buf1)
    half = EPT // 2
    nchunk = PADE // CH

    for k in range(PADE // 16 - 3, PADE // 16):
        lane = lax.broadcasted_iota(jnp.int32, (16,), 0) + k * 16
        srcf[pl.ds(k * 16, 16)] = lane - (PADE - 64)
        dstf[pl.ds(k * 16, 16)] = lane - (PADE - 64) + N

    def _load(p):
        pltpu.sync_copy(ei1.at[pl.ds(s * EPT + p * half, half)],
                        dstf.at[pl.ds(0, half)])
        pltpu.sync_copy(ei1.at[pl.ds(E + s * EPT + p * half, half)],
                        srcf.at[pl.ds(0, half)])

        def _adj(i, _):
            v = srcf[pl.ds(i * 16, 16)]
            srcf[pl.ds(i * 16, 16)] = v + off
            return 0

        lax.fori_loop(0, PADE // 16, _adj, 0)

    _load(0)
    pend = pltpu.async_copy(ycat.at[srcf.at[pl.ds(0, CH)]], buf0, gsems[0])
    pltpu.sync_copy(ycat.at[pl.ds(c * NP + s * RPT, RPT)],
                    acc_sh.at[pl.ds(s * RPT, RPT)])
    plsc.subcore_barrier()

    for p in range(2):
        if p == 1:
            _load(1)
            pend = pltpu.async_copy(ycat.at[srcf.at[pl.ds(0, CH)]], buf0,
                                    gsems[0])
        for j in range(nchunk):
            nxt = None
            if j + 1 < nchunk:
                nxt = pltpu.async_copy(
                    ycat.at[srcf.at[pl.ds((j + 1) * CH, CH)]],
                    bufs[(j + 1) % 2], gsems[(j + 1) % 2])
            pend.wait()
            pltpu.sync_copy(bufs[j % 2],
                            acc_sh.at[dstf.at[pl.ds(j * CH, CH)]], add=True)
            pend = nxt

    plsc.subcore_barrier()
    pltpu.sync_copy(acc_sh.at[pl.ds(s * RPT, RPT)],
                    acc_out.at[c, pl.ds(s * RPT, RPT), :])


_agg_kernel = functools.partial(
    pl.kernel,
    out_type=jax.ShapeDtypeStruct((2, NP, DH), jnp.float32),
    mesh=_sc_mesh,
    scratch_types=[
        pltpu.VMEM((PADE,), jnp.int32),
        pltpu.VMEM((PADE,), jnp.int32),
        pltpu.VMEM((CH, DH), jnp.float32),
        pltpu.VMEM((CH, DH), jnp.float32),
        pltpu.VMEM_SHARED((NP, DH), jnp.float32),
        pltpu.SemaphoreType.DMA,
        pltpu.SemaphoreType.DMA,
    ],
)(_agg_body)


def _lin_body(x_ref, w_ref, b_ref, deg_ref, y_ref):
    xl = lax.dot_general(x_ref[...], w_ref[...], (((1,), (1,)), ((), ())),
                         preferred_element_type=jnp.float32)
    xl = xl + b_ref[...]
    dinv = lax.rsqrt(deg_ref[0, :] + deg_ref[1, :] + 1.0)[:, None]
    y = xl * dinv
    y_ref[0] = y[:, :DH]
    y_ref[1] = y[:, DH:]


def _lin_call(x, w, b2, deg_part):
    blk = NP // 8
    return pl.pallas_call(
        _lin_body,
        grid=(8,),
        in_specs=[
            pl.BlockSpec((blk, D), lambda i: (i, 0)),
            pl.BlockSpec((D, D), lambda i: (0, 0)),
            pl.BlockSpec((1, D), lambda i: (0, 0)),
            pl.BlockSpec((2, blk), lambda i: (0, i)),
        ],
        out_specs=pl.BlockSpec((2, blk, DH), lambda i: (0, i, 0)),
        out_shape=jax.ShapeDtypeStruct((2, NP, DH), jnp.float32),
    )(x, w, b2, deg_part)


def _out_body(acc_ref, deg_ref, o_ref):
    dinv = lax.rsqrt(deg_ref[0, :] + deg_ref[1, :] + 1.0)[:, None]
    o_ref[...] = jnp.concatenate([acc_ref[0] * dinv, acc_ref[1] * dinv],
                                 axis=1)


def _out_call(acc, deg_part):
    blk = NP // 8
    return pl.pallas_call(
        _out_body,
        grid=(8,),
        in_specs=[
            pl.BlockSpec((2, blk, DH), lambda i: (0, i, 0)),
            pl.BlockSpec((2, blk), lambda i: (0, i)),
        ],
        out_specs=pl.BlockSpec((blk, D), lambda i: (i, 0)),
        out_shape=jax.ShapeDtypeStruct((N, D), jnp.float32),
    )(acc, deg_part)


def kernel(x, edge_index, W, b):
    ei1 = edge_index.reshape(2 * E)
    deg_part = _deg_kernel(ei1)
    y = _lin_call(x, W, b.reshape(1, D), deg_part)
    ycat = y.reshape(2 * NP, DH)
    acc = _agg_kernel(ei1, ycat)
    return _out_call(acc, deg_part)
